# baseline (speedup 1.0000x reference)
import functools

import jax
import jax.numpy as jnp
from jax import lax
from jax.experimental import pallas as pl
from jax.experimental.pallas import tpu as pltpu
from jax.experimental.pallas import tpu_sc as plsc

N = 50000
DIN = 128
DOUT = 32
H = 4
ALPHA = 0.2
HD = H * DOUT

NTAB = 50008
ROWS = 50048
RPT = ROWS // 16
EPAD = 409600
ER = EPAD // 128
TRASH = N

CHA = 1280
NCHA = (EPAD // 4) // CHA
CHB = 256
NCHB = (EPAD // 32) // CHB

_BLK = 1000

_mesh = plsc.VectorSubcoreMesh(core_axis_name="c", subcore_axis_name="s",
                               num_cores=2, num_subcores=16)


def _vsplat(vec, lane):
    dnums = lax.GatherDimensionNumbers(
        offset_dims=(), collapsed_slice_dims=(0,), start_index_map=(0,))
    idx = jnp.full((16,), lane, jnp.int32)
    return lax.gather(vec, idx[:, None], dnums, (1,),
                      mode=lax.GatherScatterMode.PROMISE_IN_BOUNDS)


def _proj_body(x_ref, m_ref, h_ref, p_ref, t0, t1, t2, t3):
    x = x_ref[...]
    m = m_ref[...]
    h = jnp.dot(x, m[:, :HD], preferred_element_type=jnp.float32)
    h_ref[...] = h
    p_ref[...] = jnp.dot(x, m[:, HD:], preferred_element_type=jnp.float32)
    for hh, t in enumerate((t0, t1, t2, t3)):
        t[...] = h[:, hh * DOUT:(hh + 1) * DOUT]


def _project(x, m):
    grid = (N // _BLK,)
    return pl.pallas_call(
        _proj_body,
        grid=grid,
        in_specs=[
            pl.BlockSpec((_BLK, DIN), lambda i: (i, 0)),
            pl.BlockSpec((DIN, HD + 16), lambda i: (0, 0)),
        ],
        out_specs=[
            pl.BlockSpec((_BLK, HD), lambda i: (i, 0)),
            pl.BlockSpec((_BLK, 16), lambda i: (i, 0)),
        ] + [pl.BlockSpec((_BLK, DOUT), lambda i: (i, 0))] * 4,
        out_shape=[
            jax.ShapeDtypeStruct((N, HD), jnp.float32),
            jax.ShapeDtypeStruct((N, 16), jnp.float32),
        ] + [jax.ShapeDtypeStruct((N, DOUT), jnp.float32)] * 4,
    )(x, m)


def _passA_body(adj, pst, pdt, exo,
                ps_tab, pd_tab, srcb0, srcb1, dstb0, dstb1, exb0, exb1,
                semI0, semI1, semW0, semW1):
    c = lax.axis_index("c")
    t = lax.axis_index("s")
    hh = t % 4
    q = t // 4

    pltpu.sync_copy(pst.at[c, hh, :], ps_tab)
    pltpu.sync_copy(pdt.at[c, hh, :], pd_tab)

    base_row = q * (ER // 4)
    ebase = q * (EPAD // 4)
    bufs = ((srcb0, dstb0, exb0, semI0, semW0),
            (srcb1, dstb1, exb1, semI1, semW1))

    def _idx_load(cc, sb, db, sem):
        goff = base_row + cc * 10
        pltpu.async_copy(adj.at[c, 0, pl.ds(goff, 10), :], sb, sem)
        pltpu.async_copy(adj.at[c, 1, pl.ds(goff, 10), :], db, sem)

    _idx_load(0, srcb0, dstb0, semI0)
    _idx_load(1, srcb1, dstb1, semI1)

    def pair(g, _):
        for par in range(2):
            sb, db, eb, semI, semW = bufs[par]
            cc = 2 * g + par
            pltpu.make_async_copy(adj.at[c, 0, pl.ds(base_row, 10), :], sb,
                                  semI).wait()
            pltpu.make_async_copy(adj.at[c, 1, pl.ds(base_row, 10), :], db,
                                  semI).wait()

            @pl.when(g > 0)
            def _():
                pltpu.make_async_copy(
                    eb, exo.at[c, hh, pl.ds(ebase, CHA)], semW).wait()

            def grp(j, _):
                for k in range(8):
                    iv_s = sb[j, pl.ds(k * 16, 16)]
                    iv_d = db[j, pl.ds(k * 16, 16)]
                    a = plsc.load_gather(ps_tab, [iv_s])
                    b = plsc.load_gather(pd_tab, [iv_d])
                    x = a + b
                    ex = jnp.exp(jnp.maximum(x, ALPHA * x))
                    eb[pl.ds(j * 128 + k * 16, 16)] = ex
                return 0
            lax.fori_loop(0, 10, grp, 0)

            pltpu.async_copy(eb, exo.at[c, hh, pl.ds(ebase + cc * CHA, CHA)],
                             semW)
            c2 = jnp.minimum(cc + 2, NCHA - 1)
            goff2 = base_row + c2 * 10
            pltpu.async_copy(adj.at[c, 0, pl.ds(goff2, 10), :], sb, semI)
            pltpu.async_copy(adj.at[c, 1, pl.ds(goff2, 10), :], db, semI)
        return 0
    lax.fori_loop(0, NCHA // 2, pair, 0)

    for par in range(2):
        sb, db, eb, semI, semW = bufs[par]
        pltpu.make_async_copy(eb, exo.at[c, hh, pl.ds(ebase, CHA)],
                              semW).wait()
        pltpu.make_async_copy(adj.at[c, 0, pl.ds(base_row, 10), :], sb,
                              semI).wait()
        pltpu.make_async_copy(adj.at[c, 1, pl.ds(base_row, 10), :], db,
                              semI).wait()


def _passA(adjs, pst, pdt):
    kfn = pl.kernel(
        _passA_body,
        out_type=[
            jax.ShapeDtypeStruct((2, 4, EPAD), jnp.float32),
        ],
        mesh=_mesh,
        scratch_types=[
            pltpu.VMEM((NTAB,), jnp.float32),
            pltpu.VMEM((NTAB,), jnp.float32),
            pltpu.VMEM((10, 128), jnp.int32),
            pltpu.VMEM((10, 128), jnp.int32),
            pltpu.VMEM((10, 128), jnp.int32),
            pltpu.VMEM((10, 128), jnp.int32),
            pltpu.VMEM((CHA,), jnp.float32),
            pltpu.VMEM((CHA,), jnp.float32),
            pltpu.SemaphoreType.DMA,
            pltpu.SemaphoreType.DMA,
            pltpu.SemaphoreType.DMA,
            pltpu.SemaphoreType.DMA,
        ],
        compiler_params=pltpu.CompilerParams(use_tc_tiling_on_sc=False, needs_layout_passes=False, skip_device_barrier=True),
    )
    return kfn(adjs, pst, pdt)[0]


def _passB_body(adj, exo, ht0, ht1, ht2, ht3,
                o00, o01, o02, o03, o10, o11, o12, o13, dval,
                srcb0, srcb1, dstb0, dstb1, exb0, exb1, rows0, rows1,
                acc, dsp,
                semSrc0, semSrc1, semDst0, semDst1, semEx0, semEx1,
                semG0, semG1, semS0, semS1):
    cax = lax.axis_index("c")
    sid = lax.axis_index("s")
    wid = sid * 2 + cax
    hts = (ht0, ht1, ht2, ht3)
    outs = ((o00, o01, o02, o03), (o10, o11, o12, o13))
    rbase = wid * (ER // 32)
    ebase = wid * (EPAD // 32)
    r0 = sid * RPT
    bufs = ((srcb0, dstb0, exb0, rows0, semSrc0, semDst0, semEx0, semG0, semS0),
            (srcb1, dstb1, exb1, rows1, semSrc1, semDst1, semEx1, semG1, semS1))

    for s in range(2):
        for h in range(4):
            ht = hts[h]

            def _zr(r, _):
                rows0[r, pl.ds(0, 16)] = jnp.zeros((16,), jnp.float32)
                rows0[r, pl.ds(16, 16)] = jnp.zeros((16,), jnp.float32)
                return 0
            lax.fori_loop(0, CHB, _zr, 0)

            def _ze(i, _):
                exb0[pl.ds(i * 16, 16)] = jnp.zeros((16,), jnp.float32)
                return 0
            lax.fori_loop(0, CHB // 16, _ze, 0)

            for z in range(12):
                pltpu.sync_copy(rows0, acc.at[pl.ds(r0 + z * 256, 256), :])
                pltpu.sync_copy(exb0, dsp.at[pl.ds(r0 + z * 256, 256)])
            pltpu.sync_copy(rows0.at[pl.ds(0, 56), :],
                            acc.at[pl.ds(r0 + 3072, 56), :])
            pltpu.sync_copy(exb0.at[pl.ds(0, 56)],
                            dsp.at[pl.ds(r0 + 3072, 56)])
            plsc.subcore_barrier()

            pltpu.sync_copy(adj.at[s, 0, pl.ds(rbase, 2), :], srcb0)
            pltpu.sync_copy(adj.at[s, 1, pl.ds(rbase, 2), :], dstb0)
            pltpu.sync_copy(exo.at[s, h, pl.ds(ebase, CHB)], exb0)
            for j in range(2):
                pltpu.async_copy(ht.at[srcb0.at[j]],
                                 rows0.at[pl.ds(j * 128, 128), :], semG0)
            pltpu.async_copy(adj.at[s, 0, pl.ds(rbase + 2, 2), :], srcb1,
                             semSrc1)

            def pair(g, _, s=s, h=h, ht=ht):
                for par in range(2):
                    sb, db, eb, rw, semSrc, semDst, semEx, semG, semS = bufs[par]
                    sbY, dbY, ebY, rwY, semSrcY, semDstY, semExY, semGY, semSY = bufs[1 - par]
                    cc = 2 * g + par
                    first = (par == 0)

                    for j in range(2):
                        pltpu.make_async_copy(
                            ht.at[sb.at[j]],
                            rw.at[pl.ds(j * 128, 128), :], semG).wait()

                    def _wait_exdst():
                        pltpu.make_async_copy(
                            exo.at[s, h, pl.ds(ebase, CHB)], eb, semEx).wait()
                        pltpu.make_async_copy(
                            adj.at[s, 1, pl.ds(rbase, 2), :], db,
                            semDst).wait()
                    if first:
                        pl.when(g > 0)(_wait_exdst)
                    else:
                        _wait_exdst()

                    def grp(i, _):
                        exv = eb[pl.ds(i * 16, 16)]
                        for jj in range(16):
                            e = i * 16 + jj
                            w = _vsplat(exv, jj)
                            rw[e, pl.ds(0, 16)] = rw[e, pl.ds(0, 16)] * w
                            rw[e, pl.ds(16, 16)] = rw[e, pl.ds(16, 16)] * w
                        return 0
                    lax.fori_loop(0, CHB // 16, grp, 0)

                    for j in range(2):
                        pltpu.async_copy(rw.at[pl.ds(j * 128, 128), :],
                                         acc.at[db.at[j]], semS, add=True)
                        pltpu.async_copy(eb.at[pl.ds(j * 128, 128)],
                                         dsp.at[db.at[j]], semS, add=True)

                    c2 = jnp.minimum(cc + 2, NCHB - 1)
                    pltpu.async_copy(adj.at[s, 0, pl.ds(rbase + c2 * 2, 2), :],
                                     sb, semSrc)

                    pltpu.make_async_copy(
                        adj.at[s, 0, pl.ds(rbase, 2), :], sbY, semSrcY).wait()

                    def _drain_prev():
                        for j in range(2):
                            pltpu.make_async_copy(
                                rwY.at[pl.ds(j * 128, 128), :],
                                acc.at[dbY.at[j]], semSY).wait()
                            pltpu.make_async_copy(
                                ebY.at[pl.ds(j * 128, 128)],
                                dsp.at[dbY.at[j]], semSY).wait()
                    if first:
                        pl.when(g > 0)(_drain_prev)
                    else:
                        _drain_prev()

                    c1 = jnp.minimum(cc + 1, NCHB - 1)
                    pltpu.async_copy(adj.at[s, 1, pl.ds(rbase + c1 * 2, 2), :],
                                     dbY, semDstY)
                    pltpu.async_copy(exo.at[s, h,
                                            pl.ds(ebase + c1 * CHB, CHB)],
                                     ebY, semExY)

                    for j in range(2):
                        pltpu.async_copy(ht.at[sbY.at[j]],
                                         rwY.at[pl.ds(j * 128, 128), :], semGY)
                return 0
            lax.fori_loop(0, NCHB // 2, pair, 0)

            for j in range(2):
                pltpu.make_async_copy(rows1.at[pl.ds(j * 128, 128), :],
                                      acc.at[dstb1.at[j]], semS1).wait()
                pltpu.make_async_copy(exb1.at[pl.ds(j * 128, 128)],
                                      dsp.at[dstb1.at[j]], semS1).wait()
            for j in range(2):
                pltpu.make_async_copy(ht.at[srcb0.at[j]],
                                      rows0.at[pl.ds(j * 128, 128), :],
                                      semG0).wait()
            pltpu.make_async_copy(adj.at[s, 0, pl.ds(rbase, 2), :], srcb1,
                                  semSrc1).wait()
            pltpu.make_async_copy(adj.at[s, 1, pl.ds(rbase, 2), :], dstb0,
                                  semDst0).wait()
            pltpu.make_async_copy(exo.at[s, h, pl.ds(ebase, CHB)], exb0,
                                  semEx0).wait()

            plsc.subcore_barrier()
            pltpu.sync_copy(acc.at[pl.ds(r0, RPT), :],
                            outs[s][h].at[cax, pl.ds(r0, RPT), :])
            pltpu.sync_copy(dsp.at[pl.ds(r0, RPT)],
                            dval.at[s, h, cax, pl.ds(r0, RPT)])
            plsc.subcore_barrier()


def _passB(adjs, exo, hts):
    kfn = pl.kernel(
        _passB_body,
        out_type=[jax.ShapeDtypeStruct((2, ROWS, DOUT), jnp.float32)] * 8
        + [jax.ShapeDtypeStruct((2, 4, 2, ROWS), jnp.float32)],
        mesh=_mesh,
        scratch_types=[
            pltpu.VMEM((2, 128), jnp.int32),
            pltpu.VMEM((2, 128), jnp.int32),
            pltpu.VMEM((2, 128), jnp.int32),
            pltpu.VMEM((2, 128), jnp.int32),
            pltpu.VMEM((CHB,), jnp.float32),
            pltpu.VMEM((CHB,), jnp.float32),
            pltpu.VMEM((CHB, DOUT), jnp.float32),
            pltpu.VMEM((CHB, DOUT), jnp.float32),
            pltpu.VMEM_SHARED((ROWS, DOUT), jnp.float32),
            pltpu.VMEM_SHARED((ROWS,), jnp.float32),
        ] + [pltpu.SemaphoreType.DMA] * 10,
        compiler_params=pltpu.CompilerParams(use_tc_tiling_on_sc=False, needs_layout_passes=False, skip_device_barrier=True),
    )
    res = kfn(adjs, exo, *hts)
    return res[:8], res[8]


def _final_body(*refs):
    accs = refs[0:16]
    dp_ref, dn_ref, h_ref, p_ref, r_ref = refs[16:21]
    hh_ref, hp_ref, hn_ref = refs[21:24]
    rmat = r_ref[...]

    ap = jnp.concatenate(
        [accs[2 * h][0] + accs[2 * h + 1][0] for h in range(4)], axis=-1)
    an = jnp.concatenate(
        [accs[8 + 2 * h][0] + accs[8 + 2 * h + 1][0] for h in range(4)], axis=-1)

    p = p_ref[...]
    e = p[:, 0:4] + p[:, 4:8]
    es = jnp.exp(jnp.maximum(e, ALPHA * e))
    es_rep = jnp.dot(es, rmat, preferred_element_type=jnp.float32)
    h = h_ref[...]

    dp4 = dp_ref[0][:, :, 0] + dp_ref[0][:, :, 1]
    dn4 = dn_ref[0][:, :, 0] + dn_ref[0][:, :, 1]
    dp_rep = jnp.dot(dp4, rmat, preferred_element_type=jnp.float32)
    dn_rep = jnp.dot(dn4, rmat, preferred_element_type=jnp.float32)

    hp = (ap + es_rep * h) / (dp_rep + es_rep + 1e-16)
    hn = an / (dn_rep + 1e-16)
    hh_ref[...] = jnp.maximum(hp - hn, 0.0)
    hp_ref[...] = jnp.maximum(hp, 0.0)
    hn_ref[...] = jnp.maximum(hn, 0.0)


def _final(accs8, den, h, p):
    rmat = jnp.repeat(jnp.eye(4, dtype=jnp.float32), DOUT, axis=1)
    grid = (N // _BLK,)
    in_arrays = []
    in_specs = []
    for arr in accs8:
        for cc in range(2):
            in_arrays.append(arr)
            in_specs.append(pl.BlockSpec((1, _BLK, DOUT),
                                         lambda i, cc=cc: (cc, i, 0)))
    for ss in range(2):
        in_arrays.append(den)
        in_specs.append(pl.BlockSpec((1, _BLK, 4, 2),
                                     lambda i, ss=ss: (ss, i, 0, 0)))
    in_arrays += [h, p, rmat]
    in_specs += [
        pl.BlockSpec((_BLK, HD), lambda i: (i, 0)),
        pl.BlockSpec((_BLK, 16), lambda i: (i, 0)),
        pl.BlockSpec((4, HD), lambda i: (0, 0)),
    ]
    return pl.pallas_call(
        _final_body,
        grid=grid,
        in_specs=in_specs,
        out_specs=[pl.BlockSpec((_BLK, HD), lambda i: (i, 0))] * 3,
        out_shape=[jax.ShapeDtypeStruct((N, HD), jnp.float32)] * 3,
    )(*in_arrays)


def _pad_adj(adj):
    pad = EPAD - adj.shape[1]
    filler = jnp.broadcast_to(jnp.array([[0], [TRASH]], jnp.int32), (2, pad))
    return jnp.concatenate([adj, filler], axis=1).reshape(2, ER, 128)


def _tab(x):
    return jnp.pad(x.T, ((0, 0), (0, NTAB - N)))


def kernel(node_reps, adj_pos, adj_neg, W, a_pos, a_neg):
    W2 = jnp.transpose(W, (1, 0, 2)).reshape(DIN, HD)
    eye = jnp.eye(H, dtype=jnp.float32)

    def _amat(a, half):
        return (a[:, half * DOUT:(half + 1) * DOUT][:, :, None]
                * eye[:, None, :]).reshape(HD, H)
    A = jnp.concatenate([
        _amat(a_pos, 0), _amat(a_pos, 1), _amat(a_neg, 0), _amat(a_neg, 1)
    ], axis=1)
    M = jnp.concatenate([W2, W2 @ A], axis=1)

    h, p, t0, t1, t2, t3 = _project(node_reps, M)

    pst = jnp.stack([_tab(p[:, 0:4]), _tab(p[:, 8:12])])
    pdt = jnp.stack([_tab(p[:, 4:8]), _tab(p[:, 12:16])])
    adjs = jnp.stack([_pad_adj(adj_pos), _pad_adj(adj_neg)])

    exo = _passA(adjs, pst, pdt)
    accs8, den = _passB(adjs, exo, (t0, t1, t2, t3))

    return _final(accs8, jnp.transpose(den, (0, 3, 1, 2)), h, p)


# speedup vs baseline: 1.1333x; 1.1333x over previous
import functools

import jax
import jax.numpy as jnp
from jax import lax
from jax.experimental import pallas as pl
from jax.experimental.pallas import tpu as pltpu
from jax.experimental.pallas import tpu_sc as plsc

N = 50000
DIN = 128
DOUT = 32
H = 4
ALPHA = 0.2
HD = H * DOUT

NTAB = 50008
ROWS = 50048
RPT = ROWS // 16
EPAD = 409600
ER = EPAD // 128
TRASH = N

CHA = 1280
NCHA = (EPAD // 4) // CHA
CHB = 256
NCHB = (EPAD // 32) // CHB

_BLK = 1000

_mesh = plsc.VectorSubcoreMesh(core_axis_name="c", subcore_axis_name="s",
                               num_cores=2, num_subcores=16)


def _vsplat(vec, lane):
    dnums = lax.GatherDimensionNumbers(
        offset_dims=(), collapsed_slice_dims=(0,), start_index_map=(0,))
    idx = jnp.full((16,), lane, jnp.int32)
    return lax.gather(vec, idx[:, None], dnums, (1,),
                      mode=lax.GatherScatterMode.PROMISE_IN_BOUNDS)


def _proj_body(x_ref, m_ref, h_ref, p_ref, t0, t1, t2, t3):
    x = x_ref[...]
    m = m_ref[...]
    h = jnp.dot(x, m[:, :HD], preferred_element_type=jnp.float32)
    h_ref[...] = h
    p_ref[...] = jnp.dot(x, m[:, HD:], preferred_element_type=jnp.float32)
    for hh, t in enumerate((t0, t1, t2, t3)):
        t[...] = h[:, hh * DOUT:(hh + 1) * DOUT]


def _project(x, m):
    grid = (N // _BLK,)
    return pl.pallas_call(
        _proj_body,
        grid=grid,
        in_specs=[
            pl.BlockSpec((_BLK, DIN), lambda i: (i, 0)),
            pl.BlockSpec((DIN, HD + 16), lambda i: (0, 0)),
        ],
        out_specs=[
            pl.BlockSpec((_BLK, HD), lambda i: (i, 0)),
            pl.BlockSpec((_BLK, 16), lambda i: (i, 0)),
        ] + [pl.BlockSpec((_BLK, DOUT), lambda i: (i, 0))] * 4,
        out_shape=[
            jax.ShapeDtypeStruct((N, HD), jnp.float32),
            jax.ShapeDtypeStruct((N, 16), jnp.float32),
        ] + [jax.ShapeDtypeStruct((N, DOUT), jnp.float32)] * 4,
    )(x, m)


def _passA_body(adj, pst, pdt, exo,
                ps_tab, pd_tab, srcb0, srcb1, dstb0, dstb1, exb0, exb1,
                semI0, semI1, semW0, semW1):
    c = lax.axis_index("c")
    t = lax.axis_index("s")
    hh = t % 4
    q = t // 4

    pltpu.sync_copy(pst.at[c, hh, :], ps_tab)
    pltpu.sync_copy(pdt.at[c, hh, :], pd_tab)

    base_row = q * (ER // 4)
    ebase = q * (EPAD // 4)
    bufs = ((srcb0, dstb0, exb0, semI0, semW0),
            (srcb1, dstb1, exb1, semI1, semW1))

    def _idx_load(cc, sb, db, sem):
        goff = base_row + cc * 10
        pltpu.async_copy(adj.at[c, 0, pl.ds(goff, 10), :], sb, sem)
        pltpu.async_copy(adj.at[c, 1, pl.ds(goff, 10), :], db, sem)

    _idx_load(0, srcb0, dstb0, semI0)
    _idx_load(1, srcb1, dstb1, semI1)

    def pair(g, _):
        for par in range(2):
            sb, db, eb, semI, semW = bufs[par]
            cc = 2 * g + par
            pltpu.make_async_copy(adj.at[c, 0, pl.ds(base_row, 10), :], sb,
                                  semI).wait()
            pltpu.make_async_copy(adj.at[c, 1, pl.ds(base_row, 10), :], db,
                                  semI).wait()

            @pl.when(g > 0)
            def _():
                pltpu.make_async_copy(
                    eb, exo.at[c, hh, pl.ds(ebase, CHA)], semW).wait()

            def grp(j, _):
                for k in range(8):
                    iv_s = sb[j, pl.ds(k * 16, 16)]
                    iv_d = db[j, pl.ds(k * 16, 16)]
                    a = plsc.load_gather(ps_tab, [iv_s])
                    b = plsc.load_gather(pd_tab, [iv_d])
                    x = a + b
                    ex = jnp.exp(jnp.maximum(x, ALPHA * x))
                    eb[pl.ds(j * 128 + k * 16, 16)] = ex
                return 0
            lax.fori_loop(0, 10, grp, 0)

            pltpu.async_copy(eb, exo.at[c, hh, pl.ds(ebase + cc * CHA, CHA)],
                             semW)
            c2 = jnp.minimum(cc + 2, NCHA - 1)
            goff2 = base_row + c2 * 10
            pltpu.async_copy(adj.at[c, 0, pl.ds(goff2, 10), :], sb, semI)
            pltpu.async_copy(adj.at[c, 1, pl.ds(goff2, 10), :], db, semI)
        return 0
    lax.fori_loop(0, NCHA // 2, pair, 0)

    for par in range(2):
        sb, db, eb, semI, semW = bufs[par]
        pltpu.make_async_copy(eb, exo.at[c, hh, pl.ds(ebase, CHA)],
                              semW).wait()
        pltpu.make_async_copy(adj.at[c, 0, pl.ds(base_row, 10), :], sb,
                              semI).wait()
        pltpu.make_async_copy(adj.at[c, 1, pl.ds(base_row, 10), :], db,
                              semI).wait()


def _passA(adjs, pst, pdt):
    kfn = pl.kernel(
        _passA_body,
        out_type=[
            jax.ShapeDtypeStruct((2, 4, EPAD), jnp.float32),
        ],
        mesh=_mesh,
        scratch_types=[
            pltpu.VMEM((NTAB,), jnp.float32),
            pltpu.VMEM((NTAB,), jnp.float32),
            pltpu.VMEM((10, 128), jnp.int32),
            pltpu.VMEM((10, 128), jnp.int32),
            pltpu.VMEM((10, 128), jnp.int32),
            pltpu.VMEM((10, 128), jnp.int32),
            pltpu.VMEM((CHA,), jnp.float32),
            pltpu.VMEM((CHA,), jnp.float32),
            pltpu.SemaphoreType.DMA,
            pltpu.SemaphoreType.DMA,
            pltpu.SemaphoreType.DMA,
            pltpu.SemaphoreType.DMA,
        ],
        compiler_params=pltpu.CompilerParams(use_tc_tiling_on_sc=False, needs_layout_passes=False),
    )
    return kfn(adjs, pst, pdt)[0]


def _passB_body(adj, exo, ht0, ht1, ht2, ht3,
                opos, oneg, dval,
                srcb0, srcb1, dstb0, dstb1, exb0, exb1, rows0, rows1,
                acc, dsp,
                semSrc0, semSrc1, semDst0, semDst1, semEx0, semEx1,
                semG0, semG1, semS0, semS1):
    cax = lax.axis_index("c")
    sid = lax.axis_index("s")
    wid = sid * 2 + cax
    hts = (ht0, ht1, ht2, ht3)
    outs = (opos, oneg)
    rbase = wid * (ER // 32)
    ebase = wid * (EPAD // 32)
    r0 = sid * RPT
    bufs = ((srcb0, dstb0, exb0, rows0, semSrc0, semDst0, semEx0, semG0, semS0),
            (srcb1, dstb1, exb1, rows1, semSrc1, semDst1, semEx1, semG1, semS1))

    for s in range(2):
        for h in range(4):
            ht = hts[h]

            def _zr(r, _):
                rows0[r, pl.ds(0, 16)] = jnp.zeros((16,), jnp.float32)
                rows0[r, pl.ds(16, 16)] = jnp.zeros((16,), jnp.float32)
                return 0
            lax.fori_loop(0, CHB, _zr, 0)

            def _ze(i, _):
                exb0[pl.ds(i * 16, 16)] = jnp.zeros((16,), jnp.float32)
                return 0
            lax.fori_loop(0, CHB // 16, _ze, 0)

            for z in range(12):
                pltpu.sync_copy(rows0, acc.at[pl.ds(r0 + z * 256, 256), :])
                pltpu.sync_copy(exb0, dsp.at[pl.ds(r0 + z * 256, 256)])
            pltpu.sync_copy(rows0.at[pl.ds(0, 56), :],
                            acc.at[pl.ds(r0 + 3072, 56), :])
            pltpu.sync_copy(exb0.at[pl.ds(0, 56)],
                            dsp.at[pl.ds(r0 + 3072, 56)])
            plsc.subcore_barrier()

            pltpu.sync_copy(adj.at[s, 0, pl.ds(rbase, 2), :], srcb0)
            pltpu.sync_copy(adj.at[s, 1, pl.ds(rbase, 2), :], dstb0)
            pltpu.sync_copy(exo.at[s, h, pl.ds(ebase, CHB)], exb0)
            for j in range(2):
                pltpu.async_copy(ht.at[srcb0.at[j]],
                                 rows0.at[pl.ds(j * 128, 128), :], semG0)
            pltpu.async_copy(adj.at[s, 0, pl.ds(rbase + 2, 2), :], srcb1,
                             semSrc1)

            def pair(g, _, s=s, h=h, ht=ht):
                for par in range(2):
                    sb, db, eb, rw, semSrc, semDst, semEx, semG, semS = bufs[par]
                    sbY, dbY, ebY, rwY, semSrcY, semDstY, semExY, semGY, semSY = bufs[1 - par]
                    cc = 2 * g + par
                    first = (par == 0)

                    for j in range(2):
                        pltpu.make_async_copy(
                            ht.at[sb.at[j]],
                            rw.at[pl.ds(j * 128, 128), :], semG).wait()

                    def _wait_exdst():
                        pltpu.make_async_copy(
                            exo.at[s, h, pl.ds(ebase, CHB)], eb, semEx).wait()
                        pltpu.make_async_copy(
                            adj.at[s, 1, pl.ds(rbase, 2), :], db,
                            semDst).wait()
                    if first:
                        pl.when(g > 0)(_wait_exdst)
                    else:
                        _wait_exdst()

                    def grp(i, _):
                        exv = eb[pl.ds(i * 16, 16)]
                        for jj in range(16):
                            e = i * 16 + jj
                            w = _vsplat(exv, jj)
                            rw[e, pl.ds(0, 16)] = rw[e, pl.ds(0, 16)] * w
                            rw[e, pl.ds(16, 16)] = rw[e, pl.ds(16, 16)] * w
                        return 0
                    lax.fori_loop(0, CHB // 16, grp, 0)

                    for j in range(2):
                        pltpu.async_copy(rw.at[pl.ds(j * 128, 128), :],
                                         acc.at[db.at[j]], semS, add=True)
                        pltpu.async_copy(eb.at[pl.ds(j * 128, 128)],
                                         dsp.at[db.at[j]], semS, add=True)

                    c2 = jnp.minimum(cc + 2, NCHB - 1)
                    pltpu.async_copy(adj.at[s, 0, pl.ds(rbase + c2 * 2, 2), :],
                                     sb, semSrc)

                    pltpu.make_async_copy(
                        adj.at[s, 0, pl.ds(rbase, 2), :], sbY, semSrcY).wait()

                    def _drain_prev():
                        for j in range(2):
                            pltpu.make_async_copy(
                                rwY.at[pl.ds(j * 128, 128), :],
                                acc.at[dbY.at[j]], semSY).wait()
                            pltpu.make_async_copy(
                                ebY.at[pl.ds(j * 128, 128)],
                                dsp.at[dbY.at[j]], semSY).wait()
                    if first:
                        pl.when(g > 0)(_drain_prev)
                    else:
                        _drain_prev()

                    c1 = jnp.minimum(cc + 1, NCHB - 1)
                    pltpu.async_copy(adj.at[s, 1, pl.ds(rbase + c1 * 2, 2), :],
                                     dbY, semDstY)
                    pltpu.async_copy(exo.at[s, h,
                                            pl.ds(ebase + c1 * CHB, CHB)],
                                     ebY, semExY)

                    for j in range(2):
                        pltpu.async_copy(ht.at[sbY.at[j]],
                                         rwY.at[pl.ds(j * 128, 128), :], semGY)
                return 0
            lax.fori_loop(0, NCHB // 2, pair, 0)

            for j in range(2):
                pltpu.make_async_copy(rows1.at[pl.ds(j * 128, 128), :],
                                      acc.at[dstb1.at[j]], semS1).wait()
                pltpu.make_async_copy(exb1.at[pl.ds(j * 128, 128)],
                                      dsp.at[dstb1.at[j]], semS1).wait()
            for j in range(2):
                pltpu.make_async_copy(ht.at[srcb0.at[j]],
                                      rows0.at[pl.ds(j * 128, 128), :],
                                      semG0).wait()
            pltpu.make_async_copy(adj.at[s, 0, pl.ds(rbase, 2), :], srcb1,
                                  semSrc1).wait()
            pltpu.make_async_copy(adj.at[s, 1, pl.ds(rbase, 2), :], dstb0,
                                  semDst0).wait()
            pltpu.make_async_copy(exo.at[s, h, pl.ds(ebase, CHB)], exb0,
                                  semEx0).wait()

            plsc.subcore_barrier()
            pltpu.sync_copy(acc.at[pl.ds(r0, RPT), :],
                            outs[s].at[cax, pl.ds(r0, RPT),
                                       pl.ds(h * DOUT, DOUT)])
            pltpu.sync_copy(dsp.at[pl.ds(r0, RPT)],
                            dval.at[s, h, cax, pl.ds(r0, RPT)])
            plsc.subcore_barrier()


def _passB(adjs, exo, hts):
    kfn = pl.kernel(
        _passB_body,
        out_type=[jax.ShapeDtypeStruct((2, ROWS, HD), jnp.float32)] * 2
        + [jax.ShapeDtypeStruct((2, 4, 2, ROWS), jnp.float32)],
        mesh=_mesh,
        scratch_types=[
            pltpu.VMEM((2, 128), jnp.int32),
            pltpu.VMEM((2, 128), jnp.int32),
            pltpu.VMEM((2, 128), jnp.int32),
            pltpu.VMEM((2, 128), jnp.int32),
            pltpu.VMEM((CHB,), jnp.float32),
            pltpu.VMEM((CHB,), jnp.float32),
            pltpu.VMEM((CHB, DOUT), jnp.float32),
            pltpu.VMEM((CHB, DOUT), jnp.float32),
            pltpu.VMEM_SHARED((ROWS, DOUT), jnp.float32),
            pltpu.VMEM_SHARED((ROWS,), jnp.float32),
        ] + [pltpu.SemaphoreType.DMA] * 10,
        compiler_params=pltpu.CompilerParams(use_tc_tiling_on_sc=False, needs_layout_passes=False),
    )
    res = kfn(adjs, exo, *hts)
    return res[0], res[1], res[2]


def _final_body(op0, op1, on0, on1, dp_ref, dn_ref, h_ref, p_ref, r_ref,
                hh_ref, hp_ref, hn_ref):
    rmat = r_ref[...]
    ap = op0[0] + op1[0]
    an = on0[0] + on1[0]

    p = p_ref[...]
    e = p[:, 0:4] + p[:, 4:8]
    es = jnp.exp(jnp.maximum(e, ALPHA * e))
    es_rep = jnp.dot(es, rmat, preferred_element_type=jnp.float32)
    h = h_ref[...]

    dp4 = dp_ref[0][:, :, 0] + dp_ref[0][:, :, 1]
    dn4 = dn_ref[0][:, :, 0] + dn_ref[0][:, :, 1]
    dp_rep = jnp.dot(dp4, rmat, preferred_element_type=jnp.float32)
    dn_rep = jnp.dot(dn4, rmat, preferred_element_type=jnp.float32)

    hp = (ap + es_rep * h) / (dp_rep + es_rep + 1e-16)
    hn = an / (dn_rep + 1e-16)
    hh_ref[...] = jnp.maximum(hp - hn, 0.0)
    hp_ref[...] = jnp.maximum(hp, 0.0)
    hn_ref[...] = jnp.maximum(hn, 0.0)


def _final(op, on, den, h, p):
    rmat = jnp.repeat(jnp.eye(4, dtype=jnp.float32), DOUT, axis=1)
    grid = (N // _BLK,)
    in_arrays = []
    in_specs = []
    for arr in (op, op, on, on):
        pass
    for arr, cc in ((op, 0), (op, 1), (on, 0), (on, 1)):
        in_arrays.append(arr)
        in_specs.append(pl.BlockSpec((1, _BLK, HD),
                                     lambda i, cc=cc: (cc, i, 0)))
    for ss in range(2):
        in_arrays.append(den)
        in_specs.append(pl.BlockSpec((1, _BLK, 4, 2),
                                     lambda i, ss=ss: (ss, i, 0, 0)))
    in_arrays += [h, p, rmat]
    in_specs += [
        pl.BlockSpec((_BLK, HD), lambda i: (i, 0)),
        pl.BlockSpec((_BLK, 16), lambda i: (i, 0)),
        pl.BlockSpec((4, HD), lambda i: (0, 0)),
    ]
    return pl.pallas_call(
        _final_body,
        grid=grid,
        in_specs=in_specs,
        out_specs=[pl.BlockSpec((_BLK, HD), lambda i: (i, 0))] * 3,
        out_shape=[jax.ShapeDtypeStruct((N, HD), jnp.float32)] * 3,
    )(*in_arrays)


def _pad_adj(adj):
    pad = EPAD - adj.shape[1]
    filler = jnp.broadcast_to(jnp.array([[0], [TRASH]], jnp.int32), (2, pad))
    return jnp.concatenate([adj, filler], axis=1).reshape(2, ER, 128)


def _tab(x):
    return jnp.pad(x.T, ((0, 0), (0, NTAB - N)))


def kernel(node_reps, adj_pos, adj_neg, W, a_pos, a_neg):
    W2 = jnp.transpose(W, (1, 0, 2)).reshape(DIN, HD)
    eye = jnp.eye(H, dtype=jnp.float32)

    def _amat(a, half):
        return (a[:, half * DOUT:(half + 1) * DOUT][:, :, None]
                * eye[:, None, :]).reshape(HD, H)
    A = jnp.concatenate([
        _amat(a_pos, 0), _amat(a_pos, 1), _amat(a_neg, 0), _amat(a_neg, 1)
    ], axis=1)
    M = jnp.concatenate([W2, W2 @ A], axis=1)

    h, p, t0, t1, t2, t3 = _project(node_reps, M)

    pst = jnp.stack([_tab(p[:, 0:4]), _tab(p[:, 8:12])])
    pdt = jnp.stack([_tab(p[:, 4:8]), _tab(p[:, 12:16])])
    adjs = jnp.stack([_pad_adj(adj_pos), _pad_adj(adj_neg)])

    exo = _passA(adjs, pst, pdt)
    op, on, den = _passB(adjs, exo, (t0, t1, t2, t3))

    return _final(op, on, jnp.transpose(den, (0, 3, 1, 2)), h, p)


# speedup vs baseline: 1.1386x; 1.0047x over previous
import functools

import jax
import jax.numpy as jnp
from jax import lax
from jax.experimental import pallas as pl
from jax.experimental.pallas import tpu as pltpu
from jax.experimental.pallas import tpu_sc as plsc

N = 50000
DIN = 128
DOUT = 32
H = 4
ALPHA = 0.2
HD = H * DOUT

NTAB = 50048
ROWS = 50048
RPT = ROWS // 16
EPAD = 409600
ER = EPAD // 128
TRASH = N

CHA = 1280
NCHA = (EPAD // 4) // CHA
CHB = 256
NCHB = (EPAD // 32) // CHB

_BLK = 1000

_mesh = plsc.VectorSubcoreMesh(core_axis_name="c", subcore_axis_name="s",
                               num_cores=2, num_subcores=16)


def _vsplat(vec, lane):
    dnums = lax.GatherDimensionNumbers(
        offset_dims=(), collapsed_slice_dims=(0,), start_index_map=(0,))
    idx = jnp.full((16,), lane, jnp.int32)
    return lax.gather(vec, idx[:, None], dnums, (1,),
                      mode=lax.GatherScatterMode.PROMISE_IN_BOUNDS)


def _proj_body(x_ref, m_ref, h_ref, p_ref, t0, t1, t2, t3):
    x = x_ref[...]
    m = m_ref[...]
    h = jnp.dot(x, m[:, :HD], preferred_element_type=jnp.float32)
    h_ref[...] = h
    p_ref[...] = jnp.dot(x, m[:, HD:], preferred_element_type=jnp.float32)
    for hh, t in enumerate((t0, t1, t2, t3)):
        t[...] = h[:, hh * DOUT:(hh + 1) * DOUT]


def _project(x, m):
    grid = (N // _BLK,)
    return pl.pallas_call(
        _proj_body,
        grid=grid,
        in_specs=[
            pl.BlockSpec((_BLK, DIN), lambda i: (i, 0)),
            pl.BlockSpec((DIN, HD + 16), lambda i: (0, 0)),
        ],
        out_specs=[
            pl.BlockSpec((_BLK, HD), lambda i: (i, 0)),
            pl.BlockSpec((_BLK, 16), lambda i: (i, 0)),
        ] + [pl.BlockSpec((_BLK, DOUT), lambda i: (i, 0))] * 4,
        out_shape=[
            jax.ShapeDtypeStruct((N, HD), jnp.float32),
            jax.ShapeDtypeStruct((N, 16), jnp.float32),
        ] + [jax.ShapeDtypeStruct((N, DOUT), jnp.float32)] * 4,
    )(x, m)


def _passA_body(adj, pst, pdt, exo,
                ps_tab, pd_tab, srcb0, srcb1, dstb0, dstb1, exb0, exb1,
                semI0, semI1, semW0, semW1):
    c = lax.axis_index("c")
    t = lax.axis_index("s")
    hh = t % 4
    q = t // 4

    pltpu.sync_copy(pst.at[c, hh, :], ps_tab)
    pltpu.sync_copy(pdt.at[c, hh, :], pd_tab)

    base_row = q * (ER // 4)
    ebase = q * (EPAD // 4)
    bufs = ((srcb0, dstb0, exb0, semI0, semW0),
            (srcb1, dstb1, exb1, semI1, semW1))

    def _idx_load(cc, sb, db, sem):
        goff = base_row + cc * 10
        pltpu.async_copy(adj.at[c, 0, pl.ds(goff, 10), :], sb, sem)
        pltpu.async_copy(adj.at[c, 1, pl.ds(goff, 10), :], db, sem)

    _idx_load(0, srcb0, dstb0, semI0)
    _idx_load(1, srcb1, dstb1, semI1)

    def pair(g, _):
        for par in range(2):
            sb, db, eb, semI, semW = bufs[par]
            cc = 2 * g + par
            pltpu.make_async_copy(adj.at[c, 0, pl.ds(base_row, 10), :], sb,
                                  semI).wait()
            pltpu.make_async_copy(adj.at[c, 1, pl.ds(base_row, 10), :], db,
                                  semI).wait()

            @pl.when(g > 0)
            def _():
                pltpu.make_async_copy(
                    eb, exo.at[c, hh, pl.ds(ebase, CHA)], semW).wait()

            def grp(j, _):
                for k in range(8):
                    iv_s = sb[j, pl.ds(k * 16, 16)]
                    iv_d = db[j, pl.ds(k * 16, 16)]
                    a = plsc.load_gather(ps_tab, [iv_s])
                    b = plsc.load_gather(pd_tab, [iv_d])
                    x = a + b
                    ex = jnp.exp(jnp.maximum(x, ALPHA * x))
                    eb[pl.ds(j * 128 + k * 16, 16)] = ex
                return 0
            lax.fori_loop(0, 10, grp, 0)

            pltpu.async_copy(eb, exo.at[c, hh, pl.ds(ebase + cc * CHA, CHA)],
                             semW)
            c2 = jnp.minimum(cc + 2, NCHA - 1)
            goff2 = base_row + c2 * 10
            pltpu.async_copy(adj.at[c, 0, pl.ds(goff2, 10), :], sb, semI)
            pltpu.async_copy(adj.at[c, 1, pl.ds(goff2, 10), :], db, semI)
        return 0
    lax.fori_loop(0, NCHA // 2, pair, 0)

    for par in range(2):
        sb, db, eb, semI, semW = bufs[par]
        pltpu.make_async_copy(eb, exo.at[c, hh, pl.ds(ebase, CHA)],
                              semW).wait()
        pltpu.make_async_copy(adj.at[c, 0, pl.ds(base_row, 10), :], sb,
                              semI).wait()
        pltpu.make_async_copy(adj.at[c, 1, pl.ds(base_row, 10), :], db,
                              semI).wait()


def _passA(adjs, pst, pdt):
    kfn = pl.kernel(
        _passA_body,
        out_type=[
            jax.ShapeDtypeStruct((2, 4, EPAD), jnp.float32),
        ],
        mesh=_mesh,
        scratch_types=[
            pltpu.VMEM((NTAB,), jnp.float32),
            pltpu.VMEM((NTAB,), jnp.float32),
            pltpu.VMEM((10, 128), jnp.int32),
            pltpu.VMEM((10, 128), jnp.int32),
            pltpu.VMEM((10, 128), jnp.int32),
            pltpu.VMEM((10, 128), jnp.int32),
            pltpu.VMEM((CHA,), jnp.float32),
            pltpu.VMEM((CHA,), jnp.float32),
            pltpu.SemaphoreType.DMA,
            pltpu.SemaphoreType.DMA,
            pltpu.SemaphoreType.DMA,
            pltpu.SemaphoreType.DMA,
        ],
        compiler_params=pltpu.CompilerParams(use_tc_tiling_on_sc=False, needs_layout_passes=False),
    )
    return kfn(adjs, pst, pdt)[0]


def _passB_body(adj, exo, ht0, ht1, ht2, ht3,
                opos, oneg, dval,
                srcb0, srcb1, dstb0, dstb1, exb0, exb1, rows0, rows1,
                acc, dsp,
                semSrc0, semSrc1, semDst0, semDst1, semEx0, semEx1,
                semG0, semG1, semS0, semS1):
    cax = lax.axis_index("c")
    sid = lax.axis_index("s")
    wid = sid * 2 + cax
    hts = (ht0, ht1, ht2, ht3)
    outs = (opos, oneg)
    rbase = wid * (ER // 32)
    ebase = wid * (EPAD // 32)
    r0 = sid * RPT
    bufs = ((srcb0, dstb0, exb0, rows0, semSrc0, semDst0, semEx0, semG0, semS0),
            (srcb1, dstb1, exb1, rows1, semSrc1, semDst1, semEx1, semG1, semS1))

    for s in range(2):
        for h in range(4):
            ht = hts[h]

            def _zr(r, _):
                rows0[r, pl.ds(0, 16)] = jnp.zeros((16,), jnp.float32)
                rows0[r, pl.ds(16, 16)] = jnp.zeros((16,), jnp.float32)
                return 0
            lax.fori_loop(0, CHB, _zr, 0)

            def _ze(i, _):
                exb0[pl.ds(i * 16, 16)] = jnp.zeros((16,), jnp.float32)
                return 0
            lax.fori_loop(0, CHB // 16, _ze, 0)

            for z in range(12):
                pltpu.sync_copy(rows0, acc.at[pl.ds(r0 + z * 256, 256), :])
                pltpu.sync_copy(exb0, dsp.at[pl.ds(r0 + z * 256, 256)])
            pltpu.sync_copy(rows0.at[pl.ds(0, 56), :],
                            acc.at[pl.ds(r0 + 3072, 56), :])
            pltpu.sync_copy(exb0.at[pl.ds(0, 56)],
                            dsp.at[pl.ds(r0 + 3072, 56)])
            plsc.subcore_barrier()

            pltpu.sync_copy(adj.at[s, 0, pl.ds(rbase, 2), :], srcb0)
            pltpu.sync_copy(adj.at[s, 1, pl.ds(rbase, 2), :], dstb0)
            pltpu.sync_copy(exo.at[s, h, pl.ds(ebase, CHB)], exb0)
            for j in range(2):
                pltpu.async_copy(ht.at[srcb0.at[j]],
                                 rows0.at[pl.ds(j * 128, 128), :], semG0)
            pltpu.async_copy(adj.at[s, 0, pl.ds(rbase + 2, 2), :], srcb1,
                             semSrc1)

            def pair(g, _, s=s, h=h, ht=ht):
                for par in range(2):
                    sb, db, eb, rw, semSrc, semDst, semEx, semG, semS = bufs[par]
                    sbY, dbY, ebY, rwY, semSrcY, semDstY, semExY, semGY, semSY = bufs[1 - par]
                    cc = 2 * g + par
                    first = (par == 0)

                    for j in range(2):
                        pltpu.make_async_copy(
                            ht.at[sb.at[j]],
                            rw.at[pl.ds(j * 128, 128), :], semG).wait()

                    def _wait_exdst():
                        pltpu.make_async_copy(
                            exo.at[s, h, pl.ds(ebase, CHB)], eb, semEx).wait()
                        pltpu.make_async_copy(
                            adj.at[s, 1, pl.ds(rbase, 2), :], db,
                            semDst).wait()
                    if first:
                        pl.when(g > 0)(_wait_exdst)
                    else:
                        _wait_exdst()

                    def grp(i, _):
                        exv = eb[pl.ds(i * 16, 16)]
                        for jj in range(16):
                            e = i * 16 + jj
                            w = _vsplat(exv, jj)
                            rw[e, pl.ds(0, 16)] = rw[e, pl.ds(0, 16)] * w
                            rw[e, pl.ds(16, 16)] = rw[e, pl.ds(16, 16)] * w
                        return 0
                    lax.fori_loop(0, CHB // 16, grp, 0)

                    for j in range(2):
                        pltpu.async_copy(rw.at[pl.ds(j * 128, 128), :],
                                         acc.at[db.at[j]], semS, add=True)
                        pltpu.async_copy(eb.at[pl.ds(j * 128, 128)],
                                         dsp.at[db.at[j]], semS, add=True)

                    c2 = jnp.minimum(cc + 2, NCHB - 1)
                    pltpu.async_copy(adj.at[s, 0, pl.ds(rbase + c2 * 2, 2), :],
                                     sb, semSrc)

                    pltpu.make_async_copy(
                        adj.at[s, 0, pl.ds(rbase, 2), :], sbY, semSrcY).wait()

                    def _drain_prev():
                        for j in range(2):
                            pltpu.make_async_copy(
                                rwY.at[pl.ds(j * 128, 128), :],
                                acc.at[dbY.at[j]], semSY).wait()
                            pltpu.make_async_copy(
                                ebY.at[pl.ds(j * 128, 128)],
                                dsp.at[dbY.at[j]], semSY).wait()
                    if first:
                        pl.when(g > 0)(_drain_prev)
                    else:
                        _drain_prev()

                    c1 = jnp.minimum(cc + 1, NCHB - 1)
                    pltpu.async_copy(adj.at[s, 1, pl.ds(rbase + c1 * 2, 2), :],
                                     dbY, semDstY)
                    pltpu.async_copy(exo.at[s, h,
                                            pl.ds(ebase + c1 * CHB, CHB)],
                                     ebY, semExY)

                    for j in range(2):
                        pltpu.async_copy(ht.at[sbY.at[j]],
                                         rwY.at[pl.ds(j * 128, 128), :], semGY)
                return 0
            lax.fori_loop(0, NCHB // 2, pair, 0)

            for j in range(2):
                pltpu.make_async_copy(rows1.at[pl.ds(j * 128, 128), :],
                                      acc.at[dstb1.at[j]], semS1).wait()
                pltpu.make_async_copy(exb1.at[pl.ds(j * 128, 128)],
                                      dsp.at[dstb1.at[j]], semS1).wait()
            for j in range(2):
                pltpu.make_async_copy(ht.at[srcb0.at[j]],
                                      rows0.at[pl.ds(j * 128, 128), :],
                                      semG0).wait()
            pltpu.make_async_copy(adj.at[s, 0, pl.ds(rbase, 2), :], srcb1,
                                  semSrc1).wait()
            pltpu.make_async_copy(adj.at[s, 1, pl.ds(rbase, 2), :], dstb0,
                                  semDst0).wait()
            pltpu.make_async_copy(exo.at[s, h, pl.ds(ebase, CHB)], exb0,
                                  semEx0).wait()

            plsc.subcore_barrier()
            pltpu.sync_copy(acc.at[pl.ds(r0, RPT), :],
                            outs[s].at[cax, pl.ds(r0, RPT),
                                       pl.ds(h * DOUT, DOUT)])
            pltpu.sync_copy(dsp.at[pl.ds(r0, RPT)],
                            dval.at[s, h, cax, pl.ds(r0, RPT)])
            plsc.subcore_barrier()


def _passB(adjs, exo, hts):
    kfn = pl.kernel(
        _passB_body,
        out_type=[jax.ShapeDtypeStruct((2, ROWS, HD), jnp.float32)] * 2
        + [jax.ShapeDtypeStruct((2, 4, 2, ROWS), jnp.float32)],
        mesh=_mesh,
        scratch_types=[
            pltpu.VMEM((2, 128), jnp.int32),
            pltpu.VMEM((2, 128), jnp.int32),
            pltpu.VMEM((2, 128), jnp.int32),
            pltpu.VMEM((2, 128), jnp.int32),
            pltpu.VMEM((CHB,), jnp.float32),
            pltpu.VMEM((CHB,), jnp.float32),
            pltpu.VMEM((CHB, DOUT), jnp.float32),
            pltpu.VMEM((CHB, DOUT), jnp.float32),
            pltpu.VMEM_SHARED((ROWS, DOUT), jnp.float32),
            pltpu.VMEM_SHARED((ROWS,), jnp.float32),
        ] + [pltpu.SemaphoreType.DMA] * 10,
        compiler_params=pltpu.CompilerParams(use_tc_tiling_on_sc=False, needs_layout_passes=False),
    )
    res = kfn(adjs, exo, *hts)
    return res[0], res[1], res[2]


def _final_body(op0, op1, on0, on1, dp_ref, dn_ref, h_ref, p_ref, r_ref,
                hh_ref, hp_ref, hn_ref):
    rmat = r_ref[...]
    ap = op0[0] + op1[0]
    an = on0[0] + on1[0]

    p = p_ref[...]
    e = p[:, 0:4] + p[:, 4:8]
    es = jnp.exp(jnp.maximum(e, ALPHA * e))
    es_rep = jnp.dot(es, rmat, preferred_element_type=jnp.float32)
    h = h_ref[...]

    dp4 = dp_ref[0][:, :, 0] + dp_ref[0][:, :, 1]
    dn4 = dn_ref[0][:, :, 0] + dn_ref[0][:, :, 1]
    dp_rep = jnp.dot(dp4, rmat, preferred_element_type=jnp.float32)
    dn_rep = jnp.dot(dn4, rmat, preferred_element_type=jnp.float32)

    hp = (ap + es_rep * h) / (dp_rep + es_rep + 1e-16)
    hn = an / (dn_rep + 1e-16)
    hh_ref[...] = jnp.maximum(hp - hn, 0.0)
    hp_ref[...] = jnp.maximum(hp, 0.0)
    hn_ref[...] = jnp.maximum(hn, 0.0)


def _final(op, on, den, h, p):
    rmat = jnp.repeat(jnp.eye(4, dtype=jnp.float32), DOUT, axis=1)
    grid = (N // _BLK,)
    in_arrays = []
    in_specs = []
    for arr in (op, op, on, on):
        pass
    for arr, cc in ((op, 0), (op, 1), (on, 0), (on, 1)):
        in_arrays.append(arr)
        in_specs.append(pl.BlockSpec((1, _BLK, HD),
                                     lambda i, cc=cc: (cc, i, 0)))
    for ss in range(2):
        in_arrays.append(den)
        in_specs.append(pl.BlockSpec((1, _BLK, 4, 2),
                                     lambda i, ss=ss: (ss, i, 0, 0)))
    in_arrays += [h, p, rmat]
    in_specs += [
        pl.BlockSpec((_BLK, HD), lambda i: (i, 0)),
        pl.BlockSpec((_BLK, 16), lambda i: (i, 0)),
        pl.BlockSpec((4, HD), lambda i: (0, 0)),
    ]
    return pl.pallas_call(
        _final_body,
        grid=grid,
        in_specs=in_specs,
        out_specs=[pl.BlockSpec((_BLK, HD), lambda i: (i, 0))] * 3,
        out_shape=[jax.ShapeDtypeStruct((N, HD), jnp.float32)] * 3,
    )(*in_arrays)


def _pad_adj(adj):
    pad = EPAD - adj.shape[1]
    trash = TRASH + (jnp.arange(pad, dtype=jnp.int32) % (ROWS - N))
    filler = jnp.stack([jnp.zeros((pad,), jnp.int32), trash])
    return jnp.concatenate([adj, filler], axis=1).reshape(2, ER, 128)


def _tab(x):
    return jnp.pad(x.T, ((0, 0), (0, NTAB - N)))


def kernel(node_reps, adj_pos, adj_neg, W, a_pos, a_neg):
    W2 = jnp.transpose(W, (1, 0, 2)).reshape(DIN, HD)
    eye = jnp.eye(H, dtype=jnp.float32)

    def _amat(a, half):
        return (a[:, half * DOUT:(half + 1) * DOUT][:, :, None]
                * eye[:, None, :]).reshape(HD, H)
    A = jnp.concatenate([
        _amat(a_pos, 0), _amat(a_pos, 1), _amat(a_neg, 0), _amat(a_neg, 1)
    ], axis=1)
    M = jnp.concatenate([W2, W2 @ A], axis=1)

    h, p, t0, t1, t2, t3 = _project(node_reps, M)

    pst = jnp.stack([_tab(p[:, 0:4]), _tab(p[:, 8:12])])
    pdt = jnp.stack([_tab(p[:, 4:8]), _tab(p[:, 12:16])])
    adjs = jnp.stack([_pad_adj(adj_pos), _pad_adj(adj_neg)])

    exo = _passA(adjs, pst, pdt)
    op, on, den = _passB(adjs, exo, (t0, t1, t2, t3))

    return _final(op, on, jnp.transpose(den, (0, 3, 1, 2)), h, p)
